# Initial kernel scaffold; baseline (speedup 1.0000x reference)
import jax
import jax.numpy as jnp
from jax.experimental import pallas as pl


def kernel(nfeats, edge_index, node_batch, W_fc, attn_l, attn_r, gat_bias, gn_weight, gn_bias, gn_mean_scale, cls_W, cls_b):
    raise NotImplementedError("write your pallas kernel here")


# speedup vs baseline: 20.7597x; 20.7597x over previous
import functools

import jax
import jax.numpy as jnp
from jax import lax
from jax.experimental import pallas as pl
from jax.experimental.pallas import tpu as pltpu
from jax.experimental.pallas import tpu_sc as plsc

N = 10000
E = 320000
F_IN = 128
F = 128
H = 4
B = 8
N2 = 10240
ROWW = 144
NEG_BIG = -3.0e38


def _k1_body(x_ref, w_ref, al_ref, ar_ref,
             f0_ref, f1_ref, f2_ref, f3_ref, el_ref, er_ref, mx_ref):
    i = pl.program_id(0)
    x = x_ref[...]
    fb = jnp.dot(x, w_ref[...], preferred_element_type=jnp.float32)
    els = []
    ers = []
    for h in range(H):
        fh = fb[:, h * F:(h + 1) * F]
        [f0_ref, f1_ref, f2_ref, f3_ref][h][...] = fh
        els.append(jnp.sum(fh * al_ref[h:h + 1, :], axis=1, keepdims=True))
        ers.append(jnp.sum(fh * ar_ref[h:h + 1, :], axis=1, keepdims=True))
    el = jnp.concatenate(els, axis=1)
    er = jnp.concatenate(ers, axis=1)
    el_ref[...] = el
    er_ref[...] = er
    row = jnp.concatenate(
        [jnp.max(el, axis=0, keepdims=True),
         jnp.max(er, axis=0, keepdims=True),
         jnp.full((1, 8), NEG_BIG, dtype=jnp.float32)], axis=1)

    @pl.when(i == 0)
    def _():
        mx_ref[...] = row

    @pl.when(i > 0)
    def _():
        mx_ref[...] = jnp.maximum(mx_ref[...], row)


def _k1(nfeats, W_fc, attn_l, attn_r):
    Nb = 1000
    grid = (N // Nb,)
    out_shapes = (
        [jax.ShapeDtypeStruct((N, F), jnp.float32) for _ in range(H)]
        + [jax.ShapeDtypeStruct((N, H), jnp.float32),
           jax.ShapeDtypeStruct((N, H), jnp.float32),
           jax.ShapeDtypeStruct((1, 16), jnp.float32)]
    )
    outs = pl.pallas_call(
        _k1_body,
        grid=grid,
        in_specs=[
            pl.BlockSpec((Nb, F_IN), lambda i: (i, 0)),
            pl.BlockSpec((F_IN, H * F), lambda i: (0, 0)),
            pl.BlockSpec((H, F), lambda i: (0, 0)),
            pl.BlockSpec((H, F), lambda i: (0, 0)),
        ],
        out_specs=(
            [pl.BlockSpec((Nb, F), lambda i: (i, 0)) for _ in range(H)]
            + [pl.BlockSpec((Nb, H), lambda i: (i, 0)),
               pl.BlockSpec((Nb, H), lambda i: (i, 0)),
               pl.BlockSpec((1, 16), lambda i: (0, 0))]
        ),
        out_shape=out_shapes,
    )(nfeats, W_fc, attn_l, attn_r)
    feats = outs[:H]
    el, er, mx = outs[H], outs[H + 1], outs[H + 2]
    return feats, el, er, mx


def _k3a_body(acc_ref, nb_ref, bias_ref,
              d00_ref, d01_ref, d02_ref, d03_ref,
              d10_ref, d11_ref, d12_ref, d13_ref,
              hmean_ref, sums_ref, cnt_ref):
    i = pl.program_id(0)
    d0 = [d00_ref, d01_ref, d02_ref, d03_ref]
    d1 = [d10_ref, d11_ref, d12_ref, d13_ref]
    hmean = jnp.zeros(hmean_ref.shape, jnp.float32)
    for h in range(H):
        num = acc_ref[0, h, :, :] + acc_ref[1, h, :, :]
        den = d0[h][...] + d1[h][...]
        rst = num / jnp.maximum(den, 1e-30) + bias_ref[h:h + 1, :]
        hmean = hmean + rst
    hmean = hmean * (1.0 / H)
    hmean_ref[...] = hmean
    nb = nb_ref[...]
    gids = lax.broadcasted_iota(jnp.int32, (nb.shape[0], B), 1)
    onehot = (nb == gids).astype(jnp.float32)
    dn = (((0,), (0,)), ((), ()))
    s1 = lax.dot_general(onehot, hmean, dn, preferred_element_type=jnp.float32)
    c1 = lax.dot_general(onehot, jnp.ones_like(hmean), dn,
                         preferred_element_type=jnp.float32)

    @pl.when(i == 0)
    def _():
        sums_ref[...] = s1
        cnt_ref[...] = c1

    @pl.when(i > 0)
    def _():
        sums_ref[...] = sums_ref[...] + s1
        cnt_ref[...] = cnt_ref[...] + c1


def _k3v_body(hmean_ref, nb_ref, sums_ref, cnt_ref, gs_ref, vsum_ref):
    i = pl.program_id(0)
    cs = jnp.maximum(cnt_ref[...], 1.0)
    m = sums_ref[...] / cs
    nb = nb_ref[...]
    gids = lax.broadcasted_iota(jnp.int32, (nb.shape[0], B), 1)
    onehot = (nb == gids).astype(jnp.float32)
    m_r = jnp.dot(onehot, m, preferred_element_type=jnp.float32)
    sub = hmean_ref[...] - gs_ref[...] * m_r
    dn = (((0,), (0,)), ((), ()))
    v1 = lax.dot_general(onehot, sub * sub, dn,
                         preferred_element_type=jnp.float32)

    @pl.when(i == 0)
    def _():
        vsum_ref[...] = v1

    @pl.when(i > 0)
    def _():
        vsum_ref[...] = vsum_ref[...] + v1


def _k3b_body(hmean_ref, nb_ref, sums_ref, vsum_ref, cnt_ref,
              gw_ref, gb_ref, gs_ref, cw_ref, cb_ref,
              hout_ref, hg_ref, logits_ref):
    i = pl.program_id(0)
    npr = pl.num_programs(0)
    cs = jnp.maximum(cnt_ref[...], 1.0)
    m = sums_ref[...] / cs
    s = gs_ref[...]
    var = vsum_ref[...] / cs
    nb = nb_ref[...]
    gids = lax.broadcasted_iota(jnp.int32, (nb.shape[0], B), 1)
    onehot = (nb == gids).astype(jnp.float32)
    m_r = jnp.dot(onehot, m, preferred_element_type=jnp.float32)
    v_r = jnp.dot(onehot, var, preferred_element_type=jnp.float32)
    sub = hmean_ref[...] - s * m_r
    hn = gw_ref[...] * sub / jnp.sqrt(v_r + 1e-5) + gb_ref[...]
    hn = jnp.where(hn > 0, hn, jnp.exp(jnp.minimum(hn, 0.0)) - 1.0)
    hout_ref[...] = hn
    dn = (((0,), (0,)), ((), ()))
    g1 = lax.dot_general(onehot, hn, dn, preferred_element_type=jnp.float32)

    @pl.when(i == 0)
    def _():
        hg_ref[...] = g1

    @pl.when(i > 0)
    def _():
        hg_ref[...] = hg_ref[...] + g1

    @pl.when(i == npr - 1)
    def _():
        hg = hg_ref[...] / cs
        logits_ref[...] = jnp.dot(hg, cw_ref[...],
                                  preferred_element_type=jnp.float32) + cb_ref[...]


def _k3(acc, dens, nb_pad, gat_bias, gn_weight, gn_bias, gn_mean_scale,
        cls_W, cls_b):
    Nb = 2048
    grid = (N2 // Nb,)
    hmean, sums, cnt = pl.pallas_call(
        _k3a_body,
        grid=grid,
        in_specs=[
            pl.BlockSpec((2, H, Nb, F), lambda i: (0, 0, i, 0)),
            pl.BlockSpec((Nb, 1), lambda i: (i, 0)),
            pl.BlockSpec((H, F), lambda i: (0, 0)),
        ] + [pl.BlockSpec((Nb, 1), lambda i: (i, 0)) for _ in range(8)],
        out_specs=[
            pl.BlockSpec((Nb, F), lambda i: (i, 0)),
            pl.BlockSpec((B, F), lambda i: (0, 0)),
            pl.BlockSpec((B, F), lambda i: (0, 0)),
        ],
        out_shape=[
            jax.ShapeDtypeStruct((N2, F), jnp.float32),
            jax.ShapeDtypeStruct((B, F), jnp.float32),
            jax.ShapeDtypeStruct((B, F), jnp.float32),
        ],
    )(acc, nb_pad, gat_bias, *dens)
    vsum = pl.pallas_call(
        _k3v_body,
        grid=grid,
        in_specs=[
            pl.BlockSpec((Nb, F), lambda i: (i, 0)),
            pl.BlockSpec((Nb, 1), lambda i: (i, 0)),
            pl.BlockSpec((B, F), lambda i: (0, 0)),
            pl.BlockSpec((B, F), lambda i: (0, 0)),
            pl.BlockSpec((1, F), lambda i: (0, 0)),
        ],
        out_specs=pl.BlockSpec((B, F), lambda i: (0, 0)),
        out_shape=jax.ShapeDtypeStruct((B, F), jnp.float32),
    )(hmean, nb_pad, sums, cnt, gn_mean_scale.reshape(1, F))
    hout, hg, logits = pl.pallas_call(
        _k3b_body,
        grid=grid,
        in_specs=[
            pl.BlockSpec((Nb, F), lambda i: (i, 0)),
            pl.BlockSpec((Nb, 1), lambda i: (i, 0)),
            pl.BlockSpec((B, F), lambda i: (0, 0)),
            pl.BlockSpec((B, F), lambda i: (0, 0)),
            pl.BlockSpec((B, F), lambda i: (0, 0)),
            pl.BlockSpec((1, F), lambda i: (0, 0)),
            pl.BlockSpec((1, F), lambda i: (0, 0)),
            pl.BlockSpec((1, F), lambda i: (0, 0)),
            pl.BlockSpec((F, 1), lambda i: (0, 0)),
            pl.BlockSpec((1, 1), lambda i: (0, 0)),
        ],
        out_specs=[
            pl.BlockSpec((Nb, F), lambda i: (i, 0)),
            pl.BlockSpec((B, F), lambda i: (0, 0)),
            pl.BlockSpec((B, 1), lambda i: (0, 0)),
        ],
        out_shape=[
            jax.ShapeDtypeStruct((N2, F), jnp.float32),
            jax.ShapeDtypeStruct((B, F), jnp.float32),
            jax.ShapeDtypeStruct((B, 1), jnp.float32),
        ],
    )(hmean, nb_pad, sums, vsum, cnt,
      gn_weight.reshape(1, F), gn_bias.reshape(1, F),
      gn_mean_scale.reshape(1, F), cls_W, cls_b.reshape(1, 1))
    return hout, logits


_MESH = plsc.VectorSubcoreMesh(core_axis_name="c", subcore_axis_name="s")
NW = 32
EP = E // NW
NP = N2 // NW
NPS = N2 // 16
CH = 2000
BT = 80


def _lrelu(x):
    return jnp.where(x >= 0, x, 0.2 * x)


def _wid():
    return lax.axis_index("s") * 2 + lax.axis_index("c")


def _s1_kernel(src_hbm, dst_hbm, el_hbm, er_hbm, mx_hbm, w_hbm,
               elv, erv, cv, srcv, dstv, wv):
    wid = _wid()
    base = wid * EP
    pltpu.sync_copy(el_hbm, elv)
    pltpu.sync_copy(er_hbm, erv)
    pltpu.sync_copy(mx_hbm, cv)
    mx = cv[...]
    cs = [mx[h] + mx[4 + h] for h in range(H)]
    cs = [jnp.where(x >= 0, x, 0.2 * x) for x in cs]
    for chunk in range(EP // CH):
        off = base + chunk * CH
        pltpu.sync_copy(src_hbm.at[pl.ds(off, CH)], srcv)
        pltpu.sync_copy(dst_hbm.at[pl.ds(off, CH)], dstv)

        def body(g, _):
            s16 = srcv[pl.ds(g * 16, 16)]
            d16 = dstv[pl.ds(g * 16, 16)]
            for h in range(H):
                a = plsc.load_gather(elv, [s16 * H + h])
                b = plsc.load_gather(erv, [d16 * H + h])
                e = _lrelu(a + b)
                wv[pl.ds(h * CH + g * 16, 16)] = jnp.exp(
                    jnp.maximum(e - cs[h], -80.0))
            return _

        lax.fori_loop(0, CH // 16, body, None)
        for h in range(H):
            pltpu.sync_copy(wv.at[pl.ds(h * CH, CH)],
                            w_hbm.at[pl.ds(h * E + off, CH)])


def _s1(src, dst, el, er, mx):
    k = pl.kernel(
        _s1_kernel,
        out_type=jax.ShapeDtypeStruct((H * E,), jnp.float32),
        mesh=_MESH,
        compiler_params=pltpu.CompilerParams(needs_layout_passes=False, use_tc_tiling_on_sc=False),
        scratch_types=[
            pltpu.VMEM((N * H,), jnp.float32),
            pltpu.VMEM((N * H,), jnp.float32),
            pltpu.VMEM((16,), jnp.float32),
            pltpu.VMEM((CH,), jnp.int32),
            pltpu.VMEM((CH,), jnp.int32),
            pltpu.VMEM((H * CH,), jnp.float32),
        ],
    )
    return k(src, dst, el.reshape(N * H), er.reshape(N * H), mx.reshape(16))


def _s2_kernel(f0, f1, f2, f3, src_hbm, dst_hbm, w_hbm, acc_hbm, den_hbm,
               shared_acc, shared_den,
               idxb, dstb, wb, rows,
               sw0, sw1, sw2, sw3, den8, den4, sem):
    wid = _wid()
    cid = lax.axis_index("c")
    sid = lax.axis_index("s")
    base = wid * EP
    rbase = sid * NPS
    zero16 = jnp.zeros((16,), jnp.float32)
    iota16 = lax.iota(jnp.int32, 16)

    def zrows(r, _):
        for kk in range(F // 16):
            rows[r, pl.ds(kk * 16, 16)] = zero16
        return _

    staged_w = [sw0, sw1, sw2, sw3]
    for buf in staged_w:
        for col in range(8):
            for g in range(BT // 16):
                plsc.store_scatter(buf, [g * 16 + iota16,
                                         jnp.full((16,), col, jnp.int32)],
                                   zero16)
    lax.fori_loop(0, BT, zrows, None)
    for j in range(NPS // BT):
        pltpu.sync_copy(rows, shared_acc.at[pl.ds(rbase + j * BT, BT), :])
        pltpu.sync_copy(sw0, shared_den.at[pl.ds(rbase + j * BT, BT), :])
    feats = [f0, f1, f2, f3]
    for h in range(H):
        plsc.subcore_barrier()

        def body(b, _):
            off = base + b * BT
            pltpu.sync_copy(src_hbm.at[pl.ds(off, BT)], idxb)
            pltpu.sync_copy(dst_hbm.at[pl.ds(off, BT)], dstb)
            pltpu.sync_copy(w_hbm.at[pl.ds(h * E + off, BT)], wb)
            pltpu.async_copy(feats[h].at[idxb], rows, sem).wait()

            def scale(g, _):
                w16 = wb[pl.ds(g * 16, 16)]
                for j in range(16):
                    wj = w16[j]
                    ei = g * 16 + j
                    for kk in range(F // 16):
                        rows[ei, pl.ds(kk * 16, 16)] = (
                            rows[ei, pl.ds(kk * 16, 16)] * wj)
                plsc.store_scatter(staged_w[h],
                                   [g * 16 + iota16,
                                    jnp.full((16,), h, jnp.int32)], w16)
                return _

            lax.fori_loop(0, BT // 16, scale, None)
            pltpu.sync_copy(rows, shared_acc.at[dstb], add=True)
            pltpu.sync_copy(staged_w[h], shared_den.at[dstb], add=True)
            return _

        lax.fori_loop(0, EP // BT, body, None)
        plsc.subcore_barrier()
        for j in range(NPS // BT):
            pltpu.sync_copy(
                shared_acc.at[pl.ds(rbase + j * BT, BT), :],
                acc_hbm.at[cid, h, pl.ds(rbase + j * BT, BT), :])
        if h < H - 1:
            lax.fori_loop(0, BT, zrows, None)
            for j in range(NPS // BT):
                pltpu.sync_copy(rows,
                                shared_acc.at[pl.ds(rbase + j * BT, BT), :])
    for j in range(NPS // BT):
        pltpu.sync_copy(shared_den.at[pl.ds(rbase + j * BT, BT), :], den8)
        for g in range(BT // 16):
            rows16 = g * 16 + iota16
            for h in range(H):
                col = plsc.load_gather(den8, [rows16,
                                              jnp.full((16,), h, jnp.int32)])
                den4[pl.ds(h * NPS + j * BT + g * 16, 16)] = col
    for h in range(H):
        pltpu.sync_copy(
            den4.at[pl.ds(h * NPS, NPS)],
            den_hbm.at[pl.ds(cid * H * N2 + h * N2 + rbase, NPS)])


def _s2(feats, src, dst, w):
    k = pl.kernel(
        _s2_kernel,
        out_type=(jax.ShapeDtypeStruct((2, H, N2, F), jnp.float32),
                  jax.ShapeDtypeStruct((2 * H * N2,), jnp.float32)),
        mesh=_MESH,
        compiler_params=pltpu.CompilerParams(needs_layout_passes=False, use_tc_tiling_on_sc=False),
        scratch_types=[
            pltpu.VMEM_SHARED((N2, F), jnp.float32),
            pltpu.VMEM_SHARED((N2, 8), jnp.float32),
            pltpu.VMEM((BT,), jnp.int32),
            pltpu.VMEM((BT,), jnp.int32),
            pltpu.VMEM((BT,), jnp.float32),
            pltpu.VMEM((BT, F), jnp.float32),
            pltpu.VMEM((BT, 8), jnp.float32),
            pltpu.VMEM((BT, 8), jnp.float32),
            pltpu.VMEM((BT, 8), jnp.float32),
            pltpu.VMEM((BT, 8), jnp.float32),
            pltpu.VMEM((BT, 8), jnp.float32),
            pltpu.VMEM((H * NPS,), jnp.float32),
            pltpu.SemaphoreType.DMA,
        ],
    )
    return k(feats[0], feats[1], feats[2], feats[3], src, dst, w)


def _s3_kernel(dst_hbm, w_hbm, den_hbm, attn_hbm, denv, dstv, wv, av):
    wid = _wid()
    base = wid * EP
    pltpu.sync_copy(den_hbm, denv)
    for chunk in range(EP // CH):
        off = base + chunk * CH
        pltpu.sync_copy(dst_hbm.at[pl.ds(off, CH)], dstv)
        for h in range(H):
            pltpu.sync_copy(w_hbm.at[pl.ds(h * E + off, CH)],
                            wv.at[pl.ds(h * CH, CH)])

        def body(g, _):
            d16 = dstv[pl.ds(g * 16, 16)]
            acc = jnp.zeros((16,), jnp.float32)
            for h in range(H):
                dh0 = plsc.load_gather(denv, [h * N2 + d16])
                dh1 = plsc.load_gather(denv, [H * N2 + h * N2 + d16])
                acc = acc + wv[pl.ds(h * CH + g * 16, 16)] / (dh0 + dh1)
            av[pl.ds(g * 16, 16)] = acc * (1.0 / H)
            return _

        lax.fori_loop(0, CH // 16, body, None)
        pltpu.sync_copy(av, attn_hbm.at[pl.ds(off, CH)])


def _s3(dst, w, den_flat):
    k = pl.kernel(
        _s3_kernel,
        out_type=jax.ShapeDtypeStruct((E,), jnp.float32),
        mesh=_MESH,
        compiler_params=pltpu.CompilerParams(needs_layout_passes=False, use_tc_tiling_on_sc=False),
        scratch_types=[
            pltpu.VMEM((2 * H * N2,), jnp.float32),
            pltpu.VMEM((CH,), jnp.int32),
            pltpu.VMEM((H * CH,), jnp.float32),
            pltpu.VMEM((CH,), jnp.float32),
        ],
    )
    return k(dst, w, den_flat)


def _s1_shadow(src, dst, el, er, mx):
    csum = mx[0, 0:4] + mx[0, 4:8]
    c = jnp.where(csum >= 0, csum, 0.2 * csum)
    e = el[src] + er[dst]
    e = jnp.where(e >= 0, e, 0.2 * e)
    return jnp.exp(jnp.maximum(e - c[None, :], -80.0)).T.ravel()


def _s2_shadow(feats, src, dst, w):
    feat = jnp.stack(feats, axis=1)
    wt = w.reshape(H, E).T
    num = jax.ops.segment_sum(feat[src] * wt[:, :, None], dst,
                              num_segments=N2)
    den = jax.ops.segment_sum(wt, dst, num_segments=N2)
    acc = jnp.zeros((2, H, N2, F), jnp.float32)
    acc = acc.at[0].set(num.transpose(1, 0, 2))
    denf = jnp.zeros((2, H, N2), jnp.float32)
    denf = denf.at[0].set(den.T)
    return acc, denf.ravel()


def _s3_shadow(dst, w, den_flat):
    den = den_flat.reshape(2, H, N2)
    dsum = den[0] + den[1]
    acc = jnp.zeros((E,), jnp.float32)
    for h in range(H):
        acc = acc + w[h * E:(h + 1) * E] / dsum[h][dst]
    return acc * (1.0 / H)


def kernel(nfeats, edge_index, node_batch, W_fc, attn_l, attn_r, gat_bias,
           gn_weight, gn_bias, gn_mean_scale, cls_W, cls_b):
    src = edge_index[0].astype(jnp.int32)
    dst = edge_index[1].astype(jnp.int32)
    nb_pad = jnp.pad(node_batch.astype(jnp.int32), (0, N2 - N),
                     constant_values=127).reshape(N2, 1)
    feats, el, er, mx = _k1(nfeats, W_fc,
                            attn_l.reshape(H, F), attn_r.reshape(H, F))
    w = _s1(src, dst, el, er, mx)
    acc, den_flat = _s2(feats, src, dst, w)
    dens = [den_flat[sc * H * N2 + h * N2:
                     sc * H * N2 + (h + 1) * N2].reshape(N2, 1)
            for sc in range(2) for h in range(H)]
    hout, logits = _k3(acc, dens, nb_pad, gat_bias.reshape(H, F),
                       gn_weight, gn_bias, gn_mean_scale, cls_W, cls_b)
    attn_out = _s3(dst, w, den_flat)
    return (logits, attn_out, hout[:N])


# speedup vs baseline: 50.2504x; 2.4206x over previous
import functools

import jax
import jax.numpy as jnp
from jax import lax
from jax.experimental import pallas as pl
from jax.experimental.pallas import tpu as pltpu
from jax.experimental.pallas import tpu_sc as plsc

N = 10000
E = 320000
F_IN = 128
F = 128
H = 4
B = 8
N2 = 10240
ROWW = 144
NEG_BIG = -3.0e38


def _k1_body(x_ref, w_ref, al_ref, ar_ref,
             f0_ref, f1_ref, f2_ref, f3_ref, el_ref, er_ref, mx_ref):
    i = pl.program_id(0)
    x = x_ref[...]
    fb = jnp.dot(x, w_ref[...], preferred_element_type=jnp.float32)
    els = []
    ers = []
    for h in range(H):
        fh = fb[:, h * F:(h + 1) * F]
        [f0_ref, f1_ref, f2_ref, f3_ref][h][...] = fh
        els.append(jnp.sum(fh * al_ref[h:h + 1, :], axis=1, keepdims=True))
        ers.append(jnp.sum(fh * ar_ref[h:h + 1, :], axis=1, keepdims=True))
    el = jnp.concatenate(els, axis=1)
    er = jnp.concatenate(ers, axis=1)
    el_ref[...] = el
    er_ref[...] = er
    row = jnp.concatenate(
        [jnp.max(el, axis=0, keepdims=True),
         jnp.max(er, axis=0, keepdims=True),
         jnp.full((1, 8), NEG_BIG, dtype=jnp.float32)], axis=1)

    @pl.when(i == 0)
    def _():
        mx_ref[...] = row

    @pl.when(i > 0)
    def _():
        mx_ref[...] = jnp.maximum(mx_ref[...], row)


def _k1(nfeats, W_fc, attn_l, attn_r):
    Nb = 1000
    grid = (N // Nb,)
    out_shapes = (
        [jax.ShapeDtypeStruct((N, F), jnp.float32) for _ in range(H)]
        + [jax.ShapeDtypeStruct((N, H), jnp.float32),
           jax.ShapeDtypeStruct((N, H), jnp.float32),
           jax.ShapeDtypeStruct((1, 16), jnp.float32)]
    )
    outs = pl.pallas_call(
        _k1_body,
        grid=grid,
        in_specs=[
            pl.BlockSpec((Nb, F_IN), lambda i: (i, 0)),
            pl.BlockSpec((F_IN, H * F), lambda i: (0, 0)),
            pl.BlockSpec((H, F), lambda i: (0, 0)),
            pl.BlockSpec((H, F), lambda i: (0, 0)),
        ],
        out_specs=(
            [pl.BlockSpec((Nb, F), lambda i: (i, 0)) for _ in range(H)]
            + [pl.BlockSpec((Nb, H), lambda i: (i, 0)),
               pl.BlockSpec((Nb, H), lambda i: (i, 0)),
               pl.BlockSpec((1, 16), lambda i: (0, 0))]
        ),
        out_shape=out_shapes,
    )(nfeats, W_fc, attn_l, attn_r)
    feats = outs[:H]
    el, er, mx = outs[H], outs[H + 1], outs[H + 2]
    return feats, el, er, mx


def _k3a_body(acc_ref, nb_ref, bias_ref,
              d00_ref, d01_ref, d02_ref, d03_ref,
              d10_ref, d11_ref, d12_ref, d13_ref,
              hmean_ref, sums_ref, cnt_ref):
    i = pl.program_id(0)
    d0 = [d00_ref, d01_ref, d02_ref, d03_ref]
    d1 = [d10_ref, d11_ref, d12_ref, d13_ref]
    hmean = jnp.zeros(hmean_ref.shape, jnp.float32)
    for h in range(H):
        num = acc_ref[0, h, :, :] + acc_ref[1, h, :, :]
        den = d0[h][...] + d1[h][...]
        rst = num / jnp.maximum(den, 1e-30) + bias_ref[h:h + 1, :]
        hmean = hmean + rst
    hmean = hmean * (1.0 / H)
    hmean_ref[...] = hmean
    nb = nb_ref[...]
    gids = lax.broadcasted_iota(jnp.int32, (nb.shape[0], B), 1)
    onehot = (nb == gids).astype(jnp.float32)
    dn = (((0,), (0,)), ((), ()))
    s1 = lax.dot_general(onehot, hmean, dn, preferred_element_type=jnp.float32)
    c1 = lax.dot_general(onehot, jnp.ones_like(hmean), dn,
                         preferred_element_type=jnp.float32)

    @pl.when(i == 0)
    def _():
        sums_ref[...] = s1
        cnt_ref[...] = c1

    @pl.when(i > 0)
    def _():
        sums_ref[...] = sums_ref[...] + s1
        cnt_ref[...] = cnt_ref[...] + c1


def _k3v_body(hmean_ref, nb_ref, sums_ref, cnt_ref, gs_ref, vsum_ref):
    i = pl.program_id(0)
    cs = jnp.maximum(cnt_ref[...], 1.0)
    m = sums_ref[...] / cs
    nb = nb_ref[...]
    gids = lax.broadcasted_iota(jnp.int32, (nb.shape[0], B), 1)
    onehot = (nb == gids).astype(jnp.float32)
    m_r = jnp.dot(onehot, m, preferred_element_type=jnp.float32)
    sub = hmean_ref[...] - gs_ref[...] * m_r
    dn = (((0,), (0,)), ((), ()))
    v1 = lax.dot_general(onehot, sub * sub, dn,
                         preferred_element_type=jnp.float32)

    @pl.when(i == 0)
    def _():
        vsum_ref[...] = v1

    @pl.when(i > 0)
    def _():
        vsum_ref[...] = vsum_ref[...] + v1


def _k3b_body(hmean_ref, nb_ref, sums_ref, vsum_ref, cnt_ref,
              gw_ref, gb_ref, gs_ref, cw_ref, cb_ref,
              hout_ref, hg_ref, logits_ref):
    i = pl.program_id(0)
    npr = pl.num_programs(0)
    cs = jnp.maximum(cnt_ref[...], 1.0)
    m = sums_ref[...] / cs
    s = gs_ref[...]
    var = vsum_ref[...] / cs
    nb = nb_ref[...]
    gids = lax.broadcasted_iota(jnp.int32, (nb.shape[0], B), 1)
    onehot = (nb == gids).astype(jnp.float32)
    m_r = jnp.dot(onehot, m, preferred_element_type=jnp.float32)
    v_r = jnp.dot(onehot, var, preferred_element_type=jnp.float32)
    sub = hmean_ref[...] - s * m_r
    hn = gw_ref[...] * sub / jnp.sqrt(v_r + 1e-5) + gb_ref[...]
    hn = jnp.where(hn > 0, hn, jnp.exp(jnp.minimum(hn, 0.0)) - 1.0)
    hout_ref[...] = hn
    dn = (((0,), (0,)), ((), ()))
    g1 = lax.dot_general(onehot, hn, dn, preferred_element_type=jnp.float32)

    @pl.when(i == 0)
    def _():
        hg_ref[...] = g1

    @pl.when(i > 0)
    def _():
        hg_ref[...] = hg_ref[...] + g1

    @pl.when(i == npr - 1)
    def _():
        hg = hg_ref[...] / cs
        logits_ref[...] = jnp.dot(hg, cw_ref[...],
                                  preferred_element_type=jnp.float32) + cb_ref[...]


def _k3(acc, dens, nb_pad, gat_bias, gn_weight, gn_bias, gn_mean_scale,
        cls_W, cls_b):
    Nb = 2048
    grid = (N2 // Nb,)
    hmean, sums, cnt = pl.pallas_call(
        _k3a_body,
        grid=grid,
        in_specs=[
            pl.BlockSpec((2, H, Nb, F), lambda i: (0, 0, i, 0)),
            pl.BlockSpec((Nb, 1), lambda i: (i, 0)),
            pl.BlockSpec((H, F), lambda i: (0, 0)),
        ] + [pl.BlockSpec((Nb, 1), lambda i: (i, 0)) for _ in range(8)],
        out_specs=[
            pl.BlockSpec((Nb, F), lambda i: (i, 0)),
            pl.BlockSpec((B, F), lambda i: (0, 0)),
            pl.BlockSpec((B, F), lambda i: (0, 0)),
        ],
        out_shape=[
            jax.ShapeDtypeStruct((N2, F), jnp.float32),
            jax.ShapeDtypeStruct((B, F), jnp.float32),
            jax.ShapeDtypeStruct((B, F), jnp.float32),
        ],
    )(acc, nb_pad, gat_bias, *dens)
    vsum = pl.pallas_call(
        _k3v_body,
        grid=grid,
        in_specs=[
            pl.BlockSpec((Nb, F), lambda i: (i, 0)),
            pl.BlockSpec((Nb, 1), lambda i: (i, 0)),
            pl.BlockSpec((B, F), lambda i: (0, 0)),
            pl.BlockSpec((B, F), lambda i: (0, 0)),
            pl.BlockSpec((1, F), lambda i: (0, 0)),
        ],
        out_specs=pl.BlockSpec((B, F), lambda i: (0, 0)),
        out_shape=jax.ShapeDtypeStruct((B, F), jnp.float32),
    )(hmean, nb_pad, sums, cnt, gn_mean_scale.reshape(1, F))
    hout, hg, logits = pl.pallas_call(
        _k3b_body,
        grid=grid,
        in_specs=[
            pl.BlockSpec((Nb, F), lambda i: (i, 0)),
            pl.BlockSpec((Nb, 1), lambda i: (i, 0)),
            pl.BlockSpec((B, F), lambda i: (0, 0)),
            pl.BlockSpec((B, F), lambda i: (0, 0)),
            pl.BlockSpec((B, F), lambda i: (0, 0)),
            pl.BlockSpec((1, F), lambda i: (0, 0)),
            pl.BlockSpec((1, F), lambda i: (0, 0)),
            pl.BlockSpec((1, F), lambda i: (0, 0)),
            pl.BlockSpec((F, 1), lambda i: (0, 0)),
            pl.BlockSpec((1, 1), lambda i: (0, 0)),
        ],
        out_specs=[
            pl.BlockSpec((Nb, F), lambda i: (i, 0)),
            pl.BlockSpec((B, F), lambda i: (0, 0)),
            pl.BlockSpec((B, 1), lambda i: (0, 0)),
        ],
        out_shape=[
            jax.ShapeDtypeStruct((N2, F), jnp.float32),
            jax.ShapeDtypeStruct((B, F), jnp.float32),
            jax.ShapeDtypeStruct((B, 1), jnp.float32),
        ],
    )(hmean, nb_pad, sums, vsum, cnt,
      gn_weight.reshape(1, F), gn_bias.reshape(1, F),
      gn_mean_scale.reshape(1, F), cls_W, cls_b.reshape(1, 1))
    return hout, logits


_MESH = plsc.VectorSubcoreMesh(core_axis_name="c", subcore_axis_name="s")
NW = 32
EP = E // NW
NP = N2 // NW
NPS = N2 // 16
CH = 2000
BT = 80


def _lrelu(x):
    return jnp.where(x >= 0, x, 0.2 * x)


def _wid():
    return lax.axis_index("s") * 2 + lax.axis_index("c")


def _s1_kernel(src_hbm, dst_hbm, el_hbm, er_hbm, mx_hbm, w_hbm,
               elv, erv, cv, srcv, dstv, wv):
    wid = _wid()
    base = wid * EP
    pltpu.sync_copy(el_hbm, elv)
    pltpu.sync_copy(er_hbm, erv)
    pltpu.sync_copy(mx_hbm, cv)
    mx = cv[...]
    cs = [mx[h] + mx[4 + h] for h in range(H)]
    cs = [jnp.where(x >= 0, x, 0.2 * x) for x in cs]
    for chunk in range(EP // CH):
        off = base + chunk * CH
        pltpu.sync_copy(src_hbm.at[pl.ds(off, CH)], srcv)
        pltpu.sync_copy(dst_hbm.at[pl.ds(off, CH)], dstv)

        def body(g, _):
            s16 = srcv[pl.ds(g * 16, 16)]
            d16 = dstv[pl.ds(g * 16, 16)]
            for h in range(H):
                a = plsc.load_gather(elv, [s16 * H + h])
                b = plsc.load_gather(erv, [d16 * H + h])
                e = _lrelu(a + b)
                wv[pl.ds(h * CH + g * 16, 16)] = jnp.exp(
                    jnp.maximum(e - cs[h], -80.0))
            return _

        lax.fori_loop(0, CH // 16, body, None)
        for h in range(H):
            pltpu.sync_copy(wv.at[pl.ds(h * CH, CH)],
                            w_hbm.at[pl.ds(h * E + off, CH)])


def _s1(src, dst, el, er, mx):
    k = pl.kernel(
        _s1_kernel,
        out_type=jax.ShapeDtypeStruct((H * E,), jnp.float32),
        mesh=_MESH,
        compiler_params=pltpu.CompilerParams(needs_layout_passes=False, use_tc_tiling_on_sc=False),
        scratch_types=[
            pltpu.VMEM((N * H,), jnp.float32),
            pltpu.VMEM((N * H,), jnp.float32),
            pltpu.VMEM((16,), jnp.float32),
            pltpu.VMEM((CH,), jnp.int32),
            pltpu.VMEM((CH,), jnp.int32),
            pltpu.VMEM((H * CH,), jnp.float32),
        ],
    )
    return k(src, dst, el.reshape(N * H), er.reshape(N * H), mx.reshape(16))


def _s2_kernel(f0, f1, f2, f3, src_hbm, dst_hbm, w_hbm, acc_hbm, den_hbm,
               shared_acc, shared_den,
               i0, i1, i2, i3, d0, d1, d2, d3, w0, w1, w2, w3,
               ra, rb, sw0, sw1, sw2, sw3, den8, den4,
               lsem, gsem, ssem, wsem):
    wid = _wid()
    cid = lax.axis_index("c")
    sid = lax.axis_index("s")
    base = wid * EP
    rbase = sid * NPS
    zero16 = jnp.zeros((16,), jnp.float32)
    iota16 = lax.iota(jnp.int32, 16)
    idxb = [i0, i1, i2, i3]
    dstb = [d0, d1, d2, d3]
    wbs = [w0, w1, w2, w3]
    rows2 = [ra, rb]
    staged_w = [sw0, sw1, sw2, sw3]
    feats = [f0, f1, f2, f3]
    NB = EP // BT

    def zrows(r, _):
        for kk in range(F // 16):
            ra[r, pl.ds(kk * 16, 16)] = zero16
        return _

    def zcol(buf, col):
        for g in range(BT // 16):
            plsc.store_scatter(buf, [g * 16 + iota16,
                                     jnp.full((16,), col, jnp.int32)], zero16)

    for buf in staged_w:
        for col in range(8):
            zcol(buf, col)
    lax.fori_loop(0, BT, zrows, None)
    for j in range(NPS // BT):
        pltpu.sync_copy(ra, shared_acc.at[pl.ds(rbase + j * BT, BT), :])
        pltpu.sync_copy(sw0, shared_den.at[pl.ds(rbase + j * BT, BT), :])

    for h in range(H):
        plsc.subcore_barrier()

        def off_of(b):
            return base + b * BT

        def issue_loads(seti, b):
            pltpu.async_copy(src_hbm.at[pl.ds(off_of(b), BT)], idxb[seti],
                             lsem.at[seti])
            pltpu.async_copy(dst_hbm.at[pl.ds(off_of(b), BT)], dstb[seti],
                             lsem.at[seti])
            pltpu.async_copy(w_hbm.at[pl.ds(h * E + off_of(b), BT)],
                             wbs[seti], lsem.at[seti])

        def wait_loads(seti, b):
            pltpu.make_async_copy(src_hbm.at[pl.ds(off_of(b), BT)],
                                  idxb[seti], lsem.at[seti]).wait()
            pltpu.make_async_copy(dst_hbm.at[pl.ds(off_of(b), BT)],
                                  dstb[seti], lsem.at[seti]).wait()
            pltpu.make_async_copy(w_hbm.at[pl.ds(h * E + off_of(b), BT)],
                                  wbs[seti], lsem.at[seti]).wait()

        def issue_gather(ri, seti):
            pltpu.async_copy(feats[h].at[idxb[seti]], rows2[ri], gsem.at[ri])

        def wait_gather(ri, seti):
            pltpu.make_async_copy(feats[h].at[idxb[seti]], rows2[ri],
                                  gsem.at[ri]).wait()

        def issue_scat(ri, seti):
            pltpu.async_copy(rows2[ri], shared_acc.at[dstb[seti]],
                             ssem.at[ri], add=True)
            pltpu.async_copy(staged_w[seti], shared_den.at[dstb[seti]],
                             wsem.at[seti], add=True)

        def wait_scat_rows(ri, seti):
            pltpu.make_async_copy(rows2[ri], shared_acc.at[dstb[seti]],
                                  ssem.at[ri]).wait()

        def wait_scat_w(seti):
            pltpu.make_async_copy(staged_w[seti], shared_den.at[dstb[seti]],
                                  wsem.at[seti]).wait()

        def scale(ri, seti):
            def grp(g, carry):
                w16 = wbs[seti][pl.ds(g * 16, 16)]
                for j in range(16):
                    wj = w16[j]
                    ei = g * 16 + j
                    for kk in range(F // 16):
                        rows2[ri][ei, pl.ds(kk * 16, 16)] = (
                            rows2[ri][ei, pl.ds(kk * 16, 16)] * wj)
                plsc.store_scatter(staged_w[seti],
                                   [g * 16 + iota16,
                                    jnp.full((16,), h, jnp.int32)], w16)
                return carry

            lax.fori_loop(0, BT // 16, grp, None)

        if h > 0:
            for buf in staged_w:
                zcol(buf, h - 1)
        for b in range(3):
            issue_loads(b, b)
        wait_loads(0, 0)
        issue_gather(0, 0)
        wait_loads(1, 1)
        issue_gather(1, 1)

        def quad(q, carry):
            for k in range(4):
                b = 4 * q + k
                ri = k % 2
                seti = k

                @pl.when(b + 3 <= NB - 1)
                def _():
                    if k >= 1:
                        wait_scat_w((k + 3) % 4)
                    else:
                        @pl.when(q > 0)
                        def _():
                            wait_scat_w(3)
                    issue_loads((k + 3) % 4, b + 3)

                wait_gather(ri, seti)
                scale(ri, seti)
                issue_scat(ri, seti)

                @pl.when(b + 2 <= NB - 1)
                def _():
                    wait_scat_rows(ri, seti)
                    wait_loads((k + 2) % 4, b + 2)
                    issue_gather(ri, (k + 2) % 4)
            return carry

        lax.fori_loop(0, NB // 4, quad, None)
        wait_gather(0, 0)
        scale(0, 0)
        issue_scat(0, 0)
        wait_scat_rows(1, 3)
        wait_scat_rows(0, 0)
        wait_scat_w(1)
        wait_scat_w(2)
        wait_scat_w(3)
        wait_scat_w(0)
        plsc.subcore_barrier()
        for j in range(NPS // BT):
            pltpu.sync_copy(
                shared_acc.at[pl.ds(rbase + j * BT, BT), :],
                acc_hbm.at[cid, h, pl.ds(rbase + j * BT, BT), :])
        if h < H - 1:
            lax.fori_loop(0, BT, zrows, None)
            for j in range(NPS // BT):
                pltpu.sync_copy(ra,
                                shared_acc.at[pl.ds(rbase + j * BT, BT), :])
    for j in range(NPS // BT):
        pltpu.sync_copy(shared_den.at[pl.ds(rbase + j * BT, BT), :], den8)
        for g in range(BT // 16):
            rows16 = g * 16 + iota16
            for h in range(H):
                col = plsc.load_gather(den8, [rows16,
                                              jnp.full((16,), h, jnp.int32)])
                den4[pl.ds(h * NPS + j * BT + g * 16, 16)] = col
    for h in range(H):
        pltpu.sync_copy(
            den4.at[pl.ds(h * NPS, NPS)],
            den_hbm.at[pl.ds(cid * H * N2 + h * N2 + rbase, NPS)])


def _s2(feats, src, dst, w):
    k = pl.kernel(
        _s2_kernel,
        out_type=(jax.ShapeDtypeStruct((2, H, N2, F), jnp.float32),
                  jax.ShapeDtypeStruct((2 * H * N2,), jnp.float32)),
        mesh=_MESH,
        compiler_params=pltpu.CompilerParams(needs_layout_passes=False, use_tc_tiling_on_sc=False),
        scratch_types=(
            [pltpu.VMEM_SHARED((N2, F), jnp.float32),
             pltpu.VMEM_SHARED((N2, 8), jnp.float32)]
            + [pltpu.VMEM((BT,), jnp.int32) for _ in range(8)]
            + [pltpu.VMEM((BT,), jnp.float32) for _ in range(4)]
            + [pltpu.VMEM((BT, F), jnp.float32) for _ in range(2)]
            + [pltpu.VMEM((BT, 8), jnp.float32) for _ in range(5)]
            + [pltpu.VMEM((H * NPS,), jnp.float32),
               pltpu.SemaphoreType.DMA((4,)),
               pltpu.SemaphoreType.DMA((2,)),
               pltpu.SemaphoreType.DMA((2,)),
               pltpu.SemaphoreType.DMA((4,))]
        ),
    )
    return k(feats[0], feats[1], feats[2], feats[3], src, dst, w)


def _s3_kernel(dst_hbm, w_hbm, den_hbm, attn_hbm, denv, dstv, wv, av):
    wid = _wid()
    base = wid * EP
    pltpu.sync_copy(den_hbm, denv)
    for chunk in range(EP // CH):
        off = base + chunk * CH
        pltpu.sync_copy(dst_hbm.at[pl.ds(off, CH)], dstv)
        for h in range(H):
            pltpu.sync_copy(w_hbm.at[pl.ds(h * E + off, CH)],
                            wv.at[pl.ds(h * CH, CH)])

        def body(g, _):
            d16 = dstv[pl.ds(g * 16, 16)]
            acc = jnp.zeros((16,), jnp.float32)
            for h in range(H):
                dh0 = plsc.load_gather(denv, [h * N2 + d16])
                dh1 = plsc.load_gather(denv, [H * N2 + h * N2 + d16])
                acc = acc + wv[pl.ds(h * CH + g * 16, 16)] / (dh0 + dh1)
            av[pl.ds(g * 16, 16)] = acc * (1.0 / H)
            return _

        lax.fori_loop(0, CH // 16, body, None)
        pltpu.sync_copy(av, attn_hbm.at[pl.ds(off, CH)])


def _s3(dst, w, den_flat):
    k = pl.kernel(
        _s3_kernel,
        out_type=jax.ShapeDtypeStruct((E,), jnp.float32),
        mesh=_MESH,
        compiler_params=pltpu.CompilerParams(needs_layout_passes=False, use_tc_tiling_on_sc=False),
        scratch_types=[
            pltpu.VMEM((2 * H * N2,), jnp.float32),
            pltpu.VMEM((CH,), jnp.int32),
            pltpu.VMEM((H * CH,), jnp.float32),
            pltpu.VMEM((CH,), jnp.float32),
        ],
    )
    return k(dst, w, den_flat)


def _s1_shadow(src, dst, el, er, mx):
    csum = mx[0, 0:4] + mx[0, 4:8]
    c = jnp.where(csum >= 0, csum, 0.2 * csum)
    e = el[src] + er[dst]
    e = jnp.where(e >= 0, e, 0.2 * e)
    return jnp.exp(jnp.maximum(e - c[None, :], -80.0)).T.ravel()


def _s2_shadow(feats, src, dst, w):
    feat = jnp.stack(feats, axis=1)
    wt = w.reshape(H, E).T
    num = jax.ops.segment_sum(feat[src] * wt[:, :, None], dst,
                              num_segments=N2)
    den = jax.ops.segment_sum(wt, dst, num_segments=N2)
    acc = jnp.zeros((2, H, N2, F), jnp.float32)
    acc = acc.at[0].set(num.transpose(1, 0, 2))
    denf = jnp.zeros((2, H, N2), jnp.float32)
    denf = denf.at[0].set(den.T)
    return acc, denf.ravel()


def _s3_shadow(dst, w, den_flat):
    den = den_flat.reshape(2, H, N2)
    dsum = den[0] + den[1]
    acc = jnp.zeros((E,), jnp.float32)
    for h in range(H):
        acc = acc + w[h * E:(h + 1) * E] / dsum[h][dst]
    return acc * (1.0 / H)


def kernel(nfeats, edge_index, node_batch, W_fc, attn_l, attn_r, gat_bias,
           gn_weight, gn_bias, gn_mean_scale, cls_W, cls_b):
    src = edge_index[0].astype(jnp.int32)
    dst = edge_index[1].astype(jnp.int32)
    nb_pad = jnp.pad(node_batch.astype(jnp.int32), (0, N2 - N),
                     constant_values=127).reshape(N2, 1)
    feats, el, er, mx = _k1(nfeats, W_fc,
                            attn_l.reshape(H, F), attn_r.reshape(H, F))
    w = _s1(src, dst, el, er, mx)
    acc, den_flat = _s2(feats, src, dst, w)
    dens = [den_flat[sc * H * N2 + h * N2:
                     sc * H * N2 + (h + 1) * N2].reshape(N2, 1)
            for sc in range(2) for h in range(H)]
    hout, logits = _k3(acc, dens, nb_pad, gat_bias.reshape(H, F),
                       gn_weight, gn_bias, gn_mean_scale, cls_W, cls_b)
    attn_out = _s3(dst, w, den_flat)
    return (logits, attn_out, hout[:N])
